# baseline (speedup 1.0000x reference)
import functools

import jax
import jax.numpy as jnp
from jax import lax
from jax.experimental import pallas as pl
from jax.experimental.pallas import tpu as pltpu
from jax.experimental.pallas import tpu_sc as plsc

VOCAB = 1000
EMB = 16

_NC, _NS = 2, 16
_NW = _NC * _NS
_GATHER_CHUNK = 128

BATCH_BLK = 64


def _argmax_body(x_ref, tok_ref):
    tok_ref[...] = jnp.argmax(x_ref[...], axis=-1).astype(jnp.int32)


def _argmax_tokens_tc(x):
    b, n, vocab = x.shape
    grid = b // BATCH_BLK
    return pl.pallas_call(
        _argmax_body,
        grid=(grid,),
        in_specs=[pl.BlockSpec((BATCH_BLK, n, vocab), lambda i: (i, 0, 0))],
        out_specs=pl.BlockSpec((BATCH_BLK, n), lambda i: (i, 0)),
        out_shape=jax.ShapeDtypeStruct((b, n), jnp.int32),
        compiler_params=pltpu.CompilerParams(
            dimension_semantics=("arbitrary",)
        ),
    )(x)


def _make_sc_gather(nrows):
    b_per_w = nrows // _NW
    n_full, tail = divmod(b_per_w, _GATHER_CHUNK)
    chunks = [_GATHER_CHUNK] * n_full + ([tail] if tail else [])
    mesh = plsc.VectorSubcoreMesh(core_axis_name="c", subcore_axis_name="s")

    @functools.partial(
        pl.kernel,
        mesh=mesh,
        out_type=jax.ShapeDtypeStruct((nrows, EMB), jnp.float32),
        scratch_types=[
            pltpu.VMEM((b_per_w,), jnp.int32),
            pltpu.VMEM((b_per_w, EMB), jnp.float32),
            pltpu.SemaphoreType.DMA,
        ],
        compiler_params=pltpu.CompilerParams(use_tc_tiling_on_sc=False),
    )
    def gather(table_hbm, idx_hbm, out_hbm, idx_v, rows_v, sem):
        wid = lax.axis_index("s") * _NC + lax.axis_index("c")
        base = wid * b_per_w
        pltpu.sync_copy(idx_hbm.at[pl.ds(base, b_per_w)], idx_v)
        handles = []
        off = 0
        for sz in chunks:
            handles.append(
                pltpu.async_copy(
                    table_hbm.at[idx_v.at[pl.ds(off, sz)]],
                    rows_v.at[pl.ds(off, sz)],
                    sem,
                )
            )
            off += sz
        for h in handles:
            h.wait()
        pltpu.sync_copy(rows_v, out_hbm.at[pl.ds(base, b_per_w)])

    return gather


def kernel(x, table):
    b, n, vocab = x.shape
    nrows = b * n
    tokens = _argmax_tokens_tc(x).reshape(nrows)
    out = _make_sc_gather(nrows)(table, tokens)
    return out.reshape(b, n, EMB)
